# Initial kernel scaffold; baseline (speedup 1.0000x reference)
import jax
import jax.numpy as jnp
from jax.experimental import pallas as pl


def kernel(pred, target, logit_adjustment):
    raise NotImplementedError("write your pallas kernel here")


# speedup vs baseline: 4.7863x; 4.7863x over previous
import functools

import jax
import jax.numpy as jnp
from jax.experimental import pallas as pl
from jax.experimental.pallas import tpu as pltpu

_R = 64
_K = 128
_C = _R + _R * _K
_B = 4096


def _body(x_ref, t_ref, la_ref, clone_ref, loss_ref):
    x = x_ref[...]
    t = t_ref[...]
    la = la_ref[...]

    xp = x[:, :_R]
    mp = jnp.max(xp, axis=1, keepdims=True)
    ep = jnp.exp(xp - mp)
    sp = jnp.sum(ep, axis=1, keepdims=True)
    epo_par = ep / sp

    xap = xp + la[:, :_R]
    map_ = jnp.max(xap, axis=1, keepdims=True)
    eap = jnp.exp(xap - map_)
    sap = jnp.sum(eap, axis=1, keepdims=True)
    lsea = map_ + jnp.log(sap)
    loss_part = jnp.sum((xap - lsea) * t[:, :_R])

    pieces = [epo_par]
    for i in range(_R):
        lo = _R + _K * i
        seg = x[:, lo:lo + _K]
        m = jnp.max(seg, axis=1, keepdims=True)
        e = jnp.exp(seg - m)
        s = jnp.sum(e, axis=1, keepdims=True)
        pieces.append(e * (epo_par[:, i:i + 1] / s))

        sega = seg + la[:, lo:lo + _K]
        ma = jnp.max(sega, axis=1, keepdims=True)
        ea = jnp.exp(sega - ma)
        sa = jnp.sum(ea, axis=1, keepdims=True)
        lsa = ma + jnp.log(sa)
        loss_part = loss_part + jnp.sum((sega - lsa) * t[:, lo:lo + _K])

    clone_ref[...] = jnp.concatenate(pieces, axis=1)

    @pl.when(pl.program_id(0) == 0)
    def _init():
        loss_ref[0, 0] = 0.0

    loss_ref[0, 0] += loss_part


@functools.partial(jax.jit, static_argnames=("interpret",))
def kernel(pred, target, logit_adjustment, interpret=False):
    BR = 128
    la2 = logit_adjustment.reshape(1, _C)
    clone, acc = pl.pallas_call(
        _body,
        grid=(_B // BR,),
        in_specs=[
            pl.BlockSpec((BR, _C), lambda b: (b, 0)),
            pl.BlockSpec((BR, _C), lambda b: (b, 0)),
            pl.BlockSpec((1, _C), lambda b: (0, 0)),
        ],
        out_specs=[
            pl.BlockSpec((BR, _C), lambda b: (b, 0)),
            pl.BlockSpec(memory_space=pltpu.SMEM, block_shape=(1, 1),
                         index_map=lambda b: (0, 0)),
        ],
        out_shape=[
            jax.ShapeDtypeStruct((_B, _C), jnp.float32),
            jax.ShapeDtypeStruct((1, 1), jnp.float32),
        ],
        interpret=interpret,
    )(pred, target, la2)
    loss = -acc[0, 0] / _B
    return (loss, clone)


# speedup vs baseline: 8.7899x; 1.8365x over previous
import functools

import jax
import jax.numpy as jnp
from jax.experimental import pallas as pl
from jax.experimental.pallas import tpu as pltpu

_R = 64
_K = 128
_C = _R + _R * _K
_B = 4096


def _body(x_ref, t_ref, la_ref, las_ref, clone_ref, loss_ref):
    x = x_ref[...]
    t = t_ref[...]
    la = la_ref[...]
    las = las_ref[...]
    BR = x.shape[0]

    m = jnp.max(x, axis=1, keepdims=True)

    xp = x[:, :_R] - m
    ep = jnp.exp(xp)
    sp = jnp.sum(ep, axis=1, keepdims=True)
    epo_par = ep / sp

    xap = xp + la[:, :_R]
    sap = jnp.sum(jnp.exp(xap), axis=1, keepdims=True)
    loss_part = jnp.sum((xap - jnp.log(sap)) * t[:, :_R])

    xs = x[:, _R:] - m
    ts = t[:, _R:]
    xas = xs + las
    E = jnp.exp(xs)
    Ea = jnp.exp(xas)

    clone_parts = [epo_par]
    lse_parts = []
    for g in range(_R):
        sl = slice(_K * g, _K * (g + 1))
        s = jnp.sum(E[:, sl], axis=1, keepdims=True)
        sa = jnp.sum(Ea[:, sl], axis=1, keepdims=True)
        clone_parts.append(E[:, sl] * (epo_par[:, g:g + 1] / s))
        lse_parts.append(jnp.broadcast_to(jnp.log(sa), (BR, _K)))

    lse_b = jnp.concatenate(lse_parts, axis=1)
    loss_part = loss_part + jnp.sum((xas - lse_b) * ts)
    clone_ref[...] = jnp.concatenate(clone_parts, axis=1)

    @pl.when(pl.program_id(0) == 0)
    def _init():
        loss_ref[0, 0] = 0.0

    loss_ref[0, 0] += loss_part


@functools.partial(jax.jit, static_argnames=("interpret",))
def kernel(pred, target, logit_adjustment, interpret=False):
    BR = 128
    la2 = logit_adjustment.reshape(1, _C)
    las = la2[:, _R:]
    clone, acc = pl.pallas_call(
        _body,
        grid=(_B // BR,),
        in_specs=[
            pl.BlockSpec((BR, _C), lambda b: (b, 0)),
            pl.BlockSpec((BR, _C), lambda b: (b, 0)),
            pl.BlockSpec((1, _C), lambda b: (0, 0)),
            pl.BlockSpec((1, _C - _R), lambda b: (0, 0)),
        ],
        out_specs=[
            pl.BlockSpec((BR, _C), lambda b: (b, 0)),
            pl.BlockSpec(memory_space=pltpu.SMEM, block_shape=(1, 1),
                         index_map=lambda b: (0, 0)),
        ],
        out_shape=[
            jax.ShapeDtypeStruct((_B, _C), jnp.float32),
            jax.ShapeDtypeStruct((1, 1), jnp.float32),
        ],
        interpret=interpret,
    )(pred, target, la2, las)
    loss = -acc[0, 0] / _B
    return (loss, clone)


# speedup vs baseline: 50.8793x; 5.7884x over previous
import functools

import jax
import jax.numpy as jnp
from jax.experimental import pallas as pl
from jax.experimental.pallas import tpu as pltpu

_R = 64
_K = 128
_C = _R + _R * _K
_B = 4096


def _body(x_ref, t_ref, la_ref, clone_ref, loss_ref):
    x = x_ref[...]
    t = t_ref[...]
    la = la_ref[...]
    BC = x.shape[1]

    m = jnp.max(x, axis=0, keepdims=True)
    xs = x - m
    xa = xs + la
    E = jnp.exp(xs)
    Ea = jnp.exp(xa)

    sp = jnp.sum(E[:_R], axis=0, keepdims=True)
    spa = jnp.sum(Ea[:_R], axis=0, keepdims=True)
    epo_par = E[:_R] / sp
    loss_par = jnp.sum((xa[:_R] - jnp.log(spa)) * t[:_R])

    E3 = E[_R:].reshape(_R, _K, BC)
    Ea3 = Ea[_R:].reshape(_R, _K, BC)
    s3 = jnp.sum(E3, axis=1, keepdims=True)
    sa3 = jnp.sum(Ea3, axis=1, keepdims=True)
    clone3 = E3 * (epo_par.reshape(_R, 1, BC) / s3)
    loss3 = (xa[_R:].reshape(_R, _K, BC) - jnp.log(sa3)) * t[_R:].reshape(_R, _K, BC)

    clone_ref[:_R, :] = epo_par
    clone_ref[_R:, :] = clone3.reshape(_R * _K, BC)
    loss_part = loss_par + jnp.sum(loss3)

    @pl.when(pl.program_id(0) == 0)
    def _init():
        loss_ref[0, 0] = 0.0

    loss_ref[0, 0] += loss_part


@functools.partial(jax.jit, static_argnames=("interpret",))
def kernel(pred, target, logit_adjustment, interpret=False):
    BC = 128
    xT = pred.T
    tT = target.T
    laT = logit_adjustment.reshape(_C, 1)
    cloneT, acc = pl.pallas_call(
        _body,
        grid=(_B // BC,),
        in_specs=[
            pl.BlockSpec((_C, BC), lambda b: (0, b)),
            pl.BlockSpec((_C, BC), lambda b: (0, b)),
            pl.BlockSpec((_C, 1), lambda b: (0, 0)),
        ],
        out_specs=[
            pl.BlockSpec((_C, BC), lambda b: (0, b)),
            pl.BlockSpec(memory_space=pltpu.SMEM, block_shape=(1, 1),
                         index_map=lambda b: (0, 0)),
        ],
        out_shape=[
            jax.ShapeDtypeStruct((_C, _B), jnp.float32),
            jax.ShapeDtypeStruct((1, 1), jnp.float32),
        ],
        interpret=interpret,
    )(xT, tT, laT)
    loss = -acc[0, 0] / _B
    return (loss, cloneT.T)


# speedup vs baseline: 52.7979x; 1.0377x over previous
import functools

import jax
import jax.numpy as jnp
from jax.experimental import pallas as pl
from jax.experimental.pallas import tpu as pltpu

_R = 64
_K = 128
_C = _R + _R * _K
_B = 4096


def _body(x_ref, t_ref, la_ref, ela_ref, clone_ref, loss_ref):
    x = x_ref[...]
    t = t_ref[...]
    la = la_ref[...]
    ela = ela_ref[...]
    BC = x.shape[1]

    m = jnp.max(x, axis=0, keepdims=True)
    xs = x - m
    E = jnp.exp(xs)
    Ea = E * ela

    dot_xa_t = jnp.sum((xs + la) * t)

    sp = jnp.sum(E[:_R], axis=0, keepdims=True)
    spa = jnp.sum(Ea[:_R], axis=0, keepdims=True)
    tp = jnp.sum(t[:_R], axis=0, keepdims=True)
    epo_par = E[:_R] / sp
    lse_dot = jnp.sum(jnp.log(spa) * tp)

    E3 = E[_R:].reshape(_R, _K, BC)
    Ea3 = Ea[_R:].reshape(_R, _K, BC)
    t3 = t[_R:].reshape(_R, _K, BC)
    s3 = jnp.sum(E3, axis=1, keepdims=True)
    sa3 = jnp.sum(Ea3, axis=1, keepdims=True)
    tg3 = jnp.sum(t3, axis=1, keepdims=True)
    clone3 = E3 * (epo_par.reshape(_R, 1, BC) / s3)
    lse_dot = lse_dot + jnp.sum(jnp.log(sa3) * tg3)

    clone_ref[:_R, :] = epo_par
    clone_ref[_R:, :] = clone3.reshape(_R * _K, BC)
    loss_part = dot_xa_t - lse_dot

    @pl.when(pl.program_id(0) == 0)
    def _init():
        loss_ref[0, 0] = 0.0

    loss_ref[0, 0] += loss_part


@functools.partial(jax.jit, static_argnames=("interpret",))
def kernel(pred, target, logit_adjustment, interpret=False):
    BC = 128
    xT = pred.T
    tT = target.T
    laT = logit_adjustment.reshape(_C, 1)
    elaT = jnp.exp(laT)
    cloneT, acc = pl.pallas_call(
        _body,
        grid=(_B // BC,),
        in_specs=[
            pl.BlockSpec((_C, BC), lambda b: (0, b)),
            pl.BlockSpec((_C, BC), lambda b: (0, b)),
            pl.BlockSpec((_C, 1), lambda b: (0, 0)),
            pl.BlockSpec((_C, 1), lambda b: (0, 0)),
        ],
        out_specs=[
            pl.BlockSpec((_C, BC), lambda b: (0, b)),
            pl.BlockSpec(memory_space=pltpu.SMEM, block_shape=(1, 1),
                         index_map=lambda b: (0, 0)),
        ],
        out_shape=[
            jax.ShapeDtypeStruct((_C, _B), jnp.float32),
            jax.ShapeDtypeStruct((1, 1), jnp.float32),
        ],
        interpret=interpret,
    )(xT, tT, laT, elaT)
    loss = -acc[0, 0] / _B
    return (loss, cloneT.T)
